# Initial kernel scaffold; baseline (speedup 1.0000x reference)
import jax
import jax.numpy as jnp
from jax.experimental import pallas as pl


def kernel(x, edge_index, W_neigh0, W_root0, b0, gamma0, beta0, W_neigh1, W_root1, b1, gamma1, beta1, W_neigh2, W_root2, b2):
    raise NotImplementedError("write your pallas kernel here")


# speedup vs baseline: 2.7250x; 2.7250x over previous
import functools

import jax
import jax.numpy as jnp
from jax import lax
from jax.experimental import pallas as pl
from jax.experimental.pallas import tpu as pltpu
from jax.experimental.pallas import tpu_sc as plsc

N, E, D = 10000, 320000, 128
NC, NS = 2, 16
NW = NC * NS
CB = 128
BLK = 8
CHUNKS = 80
NBLKS = CHUNKS // BLK
EPW = CHUNKS * CB
EPAD = EPW * NW
NTAB = 10112
RPT = NTAB // NS


def _sc_agg_body(with_deg, *refs):
    if with_deg:
        (h_hbm, src_hbm, dst_hbm, agg_out, hist_out,
         src_v, dst_v, rows_v, zb, agg_sh, hist_v) = refs
    else:
        (h_hbm, src_hbm, dst_hbm, agg_out,
         src_v, dst_v, rows_v, zb, agg_sh) = refs
    c = lax.axis_index("c")
    s = lax.axis_index("s")
    wid = c * NS + s
    base = s * RPT

    @pl.loop(0, 8)
    def _(i):
        for k in range(D // 16):
            zb[i, pl.ds(k * 16, 16)] = jnp.zeros((16,), jnp.float32)

    if with_deg:
        @pl.loop(0, NTAB // 16)
        def _(i):
            hist_v[pl.ds(i * 16, 16)] = jnp.zeros((16,), jnp.float32)

    @pl.loop(0, RPT // 8)
    def _(k):
        pltpu.sync_copy(zb, agg_sh.at[pl.ds(base + k * 8, 8), :])
    plsc.subcore_barrier()

    ones16 = jnp.ones((16,), jnp.float32)

    @pl.loop(0, NBLKS)
    def _(blk):
        pltpu.sync_copy(src_hbm.at[wid, pl.ds(blk * BLK, BLK), :], src_v)
        pltpu.sync_copy(dst_hbm.at[wid, pl.ds(blk * BLK, BLK), :], dst_v)

        @pl.loop(0, BLK)
        def _(j):
            pltpu.sync_copy(h_hbm.at[src_v.at[j]], rows_v)
            pltpu.sync_copy(rows_v, agg_sh.at[dst_v.at[j]], add=True)
            if with_deg:
                for k in range(CB // 16):
                    idx16 = dst_v[j, pl.ds(k * 16, 16)]
                    plsc.addupdate_scatter(hist_v, [idx16], ones16)

    plsc.subcore_barrier()

    pltpu.sync_copy(agg_sh.at[pl.ds(base, RPT), :],
                    agg_out.at[c, pl.ds(base, RPT), :])
    if with_deg:
        pltpu.sync_copy(hist_v, hist_out.at[wid])


def _make_sc_agg(with_deg):
    mesh = plsc.VectorSubcoreMesh(core_axis_name="c", subcore_axis_name="s",
                                  num_cores=NC, num_subcores=NS)
    out_type = [jax.ShapeDtypeStruct((NC, NTAB, D), jnp.float32)]
    scratch = [
        pltpu.VMEM((BLK, CB), jnp.int32),
        pltpu.VMEM((BLK, CB), jnp.int32),
        pltpu.VMEM((CB, D), jnp.float32),
        pltpu.VMEM((8, D), jnp.float32),
        pltpu.VMEM_SHARED((NTAB, D), jnp.float32),
    ]
    if with_deg:
        out_type.append(jax.ShapeDtypeStruct((NW, NTAB), jnp.float32))
        scratch.append(pltpu.VMEM((NTAB,), jnp.float32))
    return pl.kernel(
        functools.partial(_sc_agg_body, with_deg),
        out_type=tuple(out_type) if with_deg else out_type[0],
        mesh=mesh,
        scratch_types=scratch,
        compiler_params=pltpu.CompilerParams(needs_layout_passes=False),
    )


def _recip_deg(hist_ref):
    ones_col = jnp.ones((NW, 1), jnp.float32)
    deg = lax.dot_general(hist_ref[:, :N], ones_col,
                          (((0,), (0,)), ((), ())),
                          preferred_element_type=jnp.float32)
    return 1.0 / jnp.maximum(deg, 1.0)


def _dense_bn_body(p_ref, hist_ref, h_ref, wn_ref, wr_ref, b_ref, g_ref, be_ref, o_ref):
    agg = (p_ref[0, :N, :] + p_ref[1, :N, :]) * _recip_deg(hist_ref)
    pre = (jnp.dot(agg, wn_ref[...], preferred_element_type=jnp.float32)
           + jnp.dot(h_ref[...], wr_ref[...], preferred_element_type=jnp.float32)
           + b_ref[...])
    m = jnp.mean(pre, axis=0, keepdims=True)
    v = jnp.mean((pre - m) ** 2, axis=0, keepdims=True)
    hn = (pre - m) * lax.rsqrt(v + 1e-5) * g_ref[...] + be_ref[...]
    o_ref[...] = jnp.maximum(hn, 0.0)


def _dense_lsm_body(p_ref, hist_ref, h_ref, wn_ref, wr_ref, b_ref, o_ref):
    agg = (p_ref[0, :N, :] + p_ref[1, :N, :]) * _recip_deg(hist_ref)
    pre = (jnp.dot(agg, wn_ref[...], preferred_element_type=jnp.float32)
           + jnp.dot(h_ref[...], wr_ref[...], preferred_element_type=jnp.float32)
           + b_ref[...])
    mx = jnp.max(pre, axis=1, keepdims=True)
    lse = jnp.log(jnp.sum(jnp.exp(pre - mx), axis=1, keepdims=True)) + mx
    o_ref[...] = pre - lse


_dense_bn = pl.pallas_call(
    _dense_bn_body, out_shape=jax.ShapeDtypeStruct((N, D), jnp.float32))
_dense_lsm = pl.pallas_call(
    _dense_lsm_body, out_shape=jax.ShapeDtypeStruct((N, D), jnp.float32))


def kernel(x, edge_index, W_neigh0, W_root0, b0, gamma0, beta0,
           W_neigh1, W_root1, b1, gamma1, beta1, W_neigh2, W_root2, b2):
    src = edge_index[0].astype(jnp.int32)
    dst = edge_index[1].astype(jnp.int32)
    src_slab = jnp.pad(src, (0, EPAD - E)).reshape(NW, CHUNKS, CB)
    dst_slab = jnp.pad(dst, (0, EPAD - E), constant_values=N).reshape(NW, CHUNKS, CB)

    b0r, b1r, b2r = (b.reshape(1, D) for b in (b0, b1, b2))
    g0r, g1r = gamma0.reshape(1, D), gamma1.reshape(1, D)
    be0r, be1r = beta0.reshape(1, D), beta1.reshape(1, D)

    agg_deg = _make_sc_agg(True)
    agg_only = _make_sc_agg(False)

    p0, hist = agg_deg(x, src_slab, dst_slab)
    h1 = _dense_bn(p0, hist, x, W_neigh0, W_root0, b0r, g0r, be0r)
    p1 = agg_only(h1, src_slab, dst_slab)
    h2 = _dense_bn(p1, hist, h1, W_neigh1, W_root1, b1r, g1r, be1r)
    p2 = agg_only(h2, src_slab, dst_slab)
    return _dense_lsm(p2, hist, h2, W_neigh2, W_root2, b2r)


# speedup vs baseline: 2.9608x; 1.0866x over previous
import functools

import jax
import jax.numpy as jnp
from jax import lax
from jax.experimental import pallas as pl
from jax.experimental.pallas import tpu as pltpu
from jax.experimental.pallas import tpu_sc as plsc

N, E, D = 10000, 320000, 128
NC, NS = 2, 16
NW = NC * NS
CB = 128
BLK = 8
CHUNKS = 80
NBLKS = CHUNKS // BLK
EPW = CHUNKS * CB
EPAD = EPW * NW
NTAB = 10112
RPT = NTAB // NS
ZR = 32


def _sc_agg_body(with_deg, *refs):
    if with_deg:
        (h_hbm, src_hbm, dst_hbm, agg_out, hist_out,
         src_v, dst_v, rows_v, zb, agg_sh, gsem0, gsem1, zsem, hist_v) = refs
    else:
        (h_hbm, src_hbm, dst_hbm, agg_out,
         src_v, dst_v, rows_v, zb, agg_sh, gsem0, gsem1, zsem) = refs
    gsems = (gsem0, gsem1)
    c = lax.axis_index("c")
    s = lax.axis_index("s")
    wid = c * NS + s
    base = s * RPT

    @pl.loop(0, ZR)
    def _(i):
        for k in range(D // 16):
            zb[i, pl.ds(k * 16, 16)] = jnp.zeros((16,), jnp.float32)

    if with_deg:
        @pl.loop(0, NTAB // 16)
        def _(i):
            hist_v[pl.ds(i * 16, 16)] = jnp.zeros((16,), jnp.float32)

    for k in range(RPT // ZR):
        pltpu.async_copy(zb, agg_sh.at[pl.ds(base + k * ZR, ZR), :], zsem)
    pltpu.async_copy(zb.at[pl.ds(0, RPT % ZR), :],
                     agg_sh.at[pl.ds(base + (RPT // ZR) * ZR, RPT % ZR), :], zsem)
    for k in range(RPT // ZR):
        pltpu.make_async_copy(zb, agg_sh.at[pl.ds(base + k * ZR, ZR), :], zsem).wait()
    pltpu.make_async_copy(zb.at[pl.ds(0, RPT % ZR), :],
                          agg_sh.at[pl.ds(base + (RPT // ZR) * ZR, RPT % ZR), :],
                          zsem).wait()
    plsc.subcore_barrier()

    ones16 = jnp.ones((16,), jnp.float32)

    def gather(i, p):
        return (h_hbm.at[src_v.at[i]], rows_v.at[p], gsems[p])

    @pl.loop(0, NBLKS)
    def _(blk):
        pltpu.sync_copy(src_hbm.at[wid, pl.ds(blk * BLK, BLK), :], src_v)
        pltpu.sync_copy(dst_hbm.at[wid, pl.ds(blk * BLK, BLK), :], dst_v)
        pltpu.async_copy(*gather(0, 0))
        for i in range(BLK):
            if i + 1 < BLK:
                pltpu.async_copy(*gather(i + 1, (i + 1) % 2))
            pltpu.make_async_copy(*gather(i, i % 2)).wait()
            pltpu.sync_copy(rows_v.at[i % 2], agg_sh.at[dst_v.at[i]], add=True)
            if with_deg:
                for k in range(CB // 16):
                    idx16 = dst_v[i, pl.ds(k * 16, 16)]
                    plsc.addupdate_scatter(hist_v, [idx16], ones16)

    plsc.subcore_barrier()

    pltpu.sync_copy(agg_sh.at[pl.ds(base, RPT), :],
                    agg_out.at[c, pl.ds(base, RPT), :])
    if with_deg:
        pltpu.sync_copy(hist_v, hist_out.at[wid])


def _make_sc_agg(with_deg):
    mesh = plsc.VectorSubcoreMesh(core_axis_name="c", subcore_axis_name="s",
                                  num_cores=NC, num_subcores=NS)
    out_type = [jax.ShapeDtypeStruct((NC, NTAB, D), jnp.float32)]
    scratch = [
        pltpu.VMEM((BLK, CB), jnp.int32),
        pltpu.VMEM((BLK, CB), jnp.int32),
        pltpu.VMEM((2, CB, D), jnp.float32),
        pltpu.VMEM((ZR, D), jnp.float32),
        pltpu.VMEM_SHARED((NTAB, D), jnp.float32),
        pltpu.SemaphoreType.DMA,
        pltpu.SemaphoreType.DMA,
        pltpu.SemaphoreType.DMA,
    ]
    if with_deg:
        out_type.append(jax.ShapeDtypeStruct((NW, NTAB), jnp.float32))
        scratch.append(pltpu.VMEM((NTAB,), jnp.float32))
    return pl.kernel(
        functools.partial(_sc_agg_body, with_deg),
        out_type=tuple(out_type) if with_deg else out_type[0],
        mesh=mesh,
        scratch_types=scratch,
        compiler_params=pltpu.CompilerParams(needs_layout_passes=False),
    )


def _recip_deg(hist_ref):
    ones_col = jnp.ones((NW, 1), jnp.float32)
    deg = lax.dot_general(hist_ref[:, :N], ones_col,
                          (((0,), (0,)), ((), ())),
                          preferred_element_type=jnp.float32)
    return 1.0 / jnp.maximum(deg, 1.0)


def _dense_bn_body(p_ref, hist_ref, h_ref, wn_ref, wr_ref, b_ref, g_ref, be_ref, o_ref):
    agg = (p_ref[0, :N, :] + p_ref[1, :N, :]) * _recip_deg(hist_ref)
    pre = (jnp.dot(agg, wn_ref[...], preferred_element_type=jnp.float32)
           + jnp.dot(h_ref[...], wr_ref[...], preferred_element_type=jnp.float32)
           + b_ref[...])
    m = jnp.mean(pre, axis=0, keepdims=True)
    v = jnp.mean((pre - m) ** 2, axis=0, keepdims=True)
    hn = (pre - m) * lax.rsqrt(v + 1e-5) * g_ref[...] + be_ref[...]
    o_ref[...] = jnp.maximum(hn, 0.0)


def _dense_lsm_body(p_ref, hist_ref, h_ref, wn_ref, wr_ref, b_ref, o_ref):
    agg = (p_ref[0, :N, :] + p_ref[1, :N, :]) * _recip_deg(hist_ref)
    pre = (jnp.dot(agg, wn_ref[...], preferred_element_type=jnp.float32)
           + jnp.dot(h_ref[...], wr_ref[...], preferred_element_type=jnp.float32)
           + b_ref[...])
    mx = jnp.max(pre, axis=1, keepdims=True)
    lse = jnp.log(jnp.sum(jnp.exp(pre - mx), axis=1, keepdims=True)) + mx
    o_ref[...] = pre - lse


_dense_bn = pl.pallas_call(
    _dense_bn_body, out_shape=jax.ShapeDtypeStruct((N, D), jnp.float32))
_dense_lsm = pl.pallas_call(
    _dense_lsm_body, out_shape=jax.ShapeDtypeStruct((N, D), jnp.float32))


def kernel(x, edge_index, W_neigh0, W_root0, b0, gamma0, beta0,
           W_neigh1, W_root1, b1, gamma1, beta1, W_neigh2, W_root2, b2):
    src = edge_index[0].astype(jnp.int32)
    dst = edge_index[1].astype(jnp.int32)
    src_slab = jnp.pad(src, (0, EPAD - E)).reshape(NW, CHUNKS, CB)
    dst_slab = jnp.pad(dst, (0, EPAD - E), constant_values=N).reshape(NW, CHUNKS, CB)

    b0r, b1r, b2r = (b.reshape(1, D) for b in (b0, b1, b2))
    g0r, g1r = gamma0.reshape(1, D), gamma1.reshape(1, D)
    be0r, be1r = beta0.reshape(1, D), beta1.reshape(1, D)

    agg_deg = _make_sc_agg(True)
    agg_only = _make_sc_agg(False)

    p0, hist = agg_deg(x, src_slab, dst_slab)
    h1 = _dense_bn(p0, hist, x, W_neigh0, W_root0, b0r, g0r, be0r)
    p1 = agg_only(h1, src_slab, dst_slab)
    h2 = _dense_bn(p1, hist, h1, W_neigh1, W_root1, b1r, g1r, be1r)
    p2 = agg_only(h2, src_slab, dst_slab)
    return _dense_lsm(p2, hist, h2, W_neigh2, W_root2, b2r)


# speedup vs baseline: 2.9663x; 1.0018x over previous
import functools

import jax
import jax.numpy as jnp
from jax import lax
from jax.experimental import pallas as pl
from jax.experimental.pallas import tpu as pltpu
from jax.experimental.pallas import tpu_sc as plsc

N, E, D = 10000, 320000, 128
NC, NS = 2, 16
NW = NC * NS
CB = 128
BLK = 8
CHUNKS = 80
NBLKS = CHUNKS // BLK
EPW = CHUNKS * CB
EPAD = EPW * NW
NTAB = 10112
RPT = NTAB // NS
ZR = 32


def _sc_agg_body(with_deg, *refs):
    if with_deg:
        (h_hbm, src_hbm, dst_hbm, agg_out, hist_out,
         src_v, dst_v, rows_v, zb, agg_sh, gsem0, gsem1, ssem0, ssem1, zsem,
         hist_v) = refs
    else:
        (h_hbm, src_hbm, dst_hbm, agg_out,
         src_v, dst_v, rows_v, zb, agg_sh, gsem0, gsem1, ssem0, ssem1,
         zsem) = refs
    gsems = (gsem0, gsem1)
    ssems = (ssem0, ssem1)
    c = lax.axis_index("c")
    s = lax.axis_index("s")
    wid = c * NS + s
    base = s * RPT

    @pl.loop(0, ZR)
    def _(i):
        for k in range(D // 16):
            zb[i, pl.ds(k * 16, 16)] = jnp.zeros((16,), jnp.float32)

    if with_deg:
        @pl.loop(0, NTAB // 16)
        def _(i):
            hist_v[pl.ds(i * 16, 16)] = jnp.zeros((16,), jnp.float32)

    for k in range(RPT // ZR):
        pltpu.async_copy(zb, agg_sh.at[pl.ds(base + k * ZR, ZR), :], zsem)
    pltpu.async_copy(zb.at[pl.ds(0, RPT % ZR), :],
                     agg_sh.at[pl.ds(base + (RPT // ZR) * ZR, RPT % ZR), :], zsem)
    for k in range(RPT // ZR):
        pltpu.make_async_copy(zb, agg_sh.at[pl.ds(base + k * ZR, ZR), :], zsem).wait()
    pltpu.make_async_copy(zb.at[pl.ds(0, RPT % ZR), :],
                          agg_sh.at[pl.ds(base + (RPT // ZR) * ZR, RPT % ZR), :],
                          zsem).wait()
    plsc.subcore_barrier()

    ones16 = jnp.ones((16,), jnp.float32)

    def gather(i, p):
        return (h_hbm.at[src_v.at[i]], rows_v.at[p], gsems[p])

    def scatter(i, p):
        return (rows_v.at[p], agg_sh.at[dst_v.at[i]], ssems[p])

    @pl.loop(0, NBLKS)
    def _(blk):
        pltpu.sync_copy(src_hbm.at[wid, pl.ds(blk * BLK, BLK), :], src_v)
        pltpu.sync_copy(dst_hbm.at[wid, pl.ds(blk * BLK, BLK), :], dst_v)
        pltpu.async_copy(*gather(0, 0))
        for i in range(BLK):
            p = i % 2
            if i + 1 < BLK:
                if i >= 1:
                    pltpu.make_async_copy(*scatter(i - 1, 1 - p)).wait()
                pltpu.async_copy(*gather(i + 1, 1 - p))
            pltpu.make_async_copy(*gather(i, p)).wait()
            pltpu.async_copy(*scatter(i, p), add=True)
            if with_deg:
                for k in range(CB // 16):
                    idx16 = dst_v[i, pl.ds(k * 16, 16)]
                    plsc.addupdate_scatter(hist_v, [idx16], ones16)
        pltpu.make_async_copy(*scatter(BLK - 2, 0)).wait()
        pltpu.make_async_copy(*scatter(BLK - 1, 1)).wait()

    plsc.subcore_barrier()

    pltpu.sync_copy(agg_sh.at[pl.ds(base, RPT), :],
                    agg_out.at[c, pl.ds(base, RPT), :])
    if with_deg:
        pltpu.sync_copy(hist_v, hist_out.at[wid])


def _make_sc_agg(with_deg):
    mesh = plsc.VectorSubcoreMesh(core_axis_name="c", subcore_axis_name="s",
                                  num_cores=NC, num_subcores=NS)
    out_type = [jax.ShapeDtypeStruct((NC, NTAB, D), jnp.float32)]
    scratch = [
        pltpu.VMEM((BLK, CB), jnp.int32),
        pltpu.VMEM((BLK, CB), jnp.int32),
        pltpu.VMEM((2, CB, D), jnp.float32),
        pltpu.VMEM((ZR, D), jnp.float32),
        pltpu.VMEM_SHARED((NTAB, D), jnp.float32),
        pltpu.SemaphoreType.DMA,
        pltpu.SemaphoreType.DMA,
        pltpu.SemaphoreType.DMA,
        pltpu.SemaphoreType.DMA,
        pltpu.SemaphoreType.DMA,
    ]
    if with_deg:
        out_type.append(jax.ShapeDtypeStruct((NW, NTAB), jnp.float32))
        scratch.append(pltpu.VMEM((NTAB,), jnp.float32))
    return pl.kernel(
        functools.partial(_sc_agg_body, with_deg),
        out_type=tuple(out_type) if with_deg else out_type[0],
        mesh=mesh,
        scratch_types=scratch,
        compiler_params=pltpu.CompilerParams(needs_layout_passes=False),
    )


def _recip_deg(hist_ref):
    ones_col = jnp.ones((NW, 1), jnp.float32)
    deg = lax.dot_general(hist_ref[:, :N], ones_col,
                          (((0,), (0,)), ((), ())),
                          preferred_element_type=jnp.float32)
    return 1.0 / jnp.maximum(deg, 1.0)


def _dense_bn_body(p_ref, hist_ref, h_ref, wn_ref, wr_ref, b_ref, g_ref, be_ref, o_ref):
    agg = (p_ref[0, :N, :] + p_ref[1, :N, :]) * _recip_deg(hist_ref)
    pre = (jnp.dot(agg, wn_ref[...], preferred_element_type=jnp.float32)
           + jnp.dot(h_ref[...], wr_ref[...], preferred_element_type=jnp.float32)
           + b_ref[...])
    m = jnp.mean(pre, axis=0, keepdims=True)
    v = jnp.mean((pre - m) ** 2, axis=0, keepdims=True)
    hn = (pre - m) * lax.rsqrt(v + 1e-5) * g_ref[...] + be_ref[...]
    o_ref[...] = jnp.maximum(hn, 0.0)


def _dense_lsm_body(p_ref, hist_ref, h_ref, wn_ref, wr_ref, b_ref, o_ref):
    agg = (p_ref[0, :N, :] + p_ref[1, :N, :]) * _recip_deg(hist_ref)
    pre = (jnp.dot(agg, wn_ref[...], preferred_element_type=jnp.float32)
           + jnp.dot(h_ref[...], wr_ref[...], preferred_element_type=jnp.float32)
           + b_ref[...])
    mx = jnp.max(pre, axis=1, keepdims=True)
    lse = jnp.log(jnp.sum(jnp.exp(pre - mx), axis=1, keepdims=True)) + mx
    o_ref[...] = pre - lse


_dense_bn = pl.pallas_call(
    _dense_bn_body, out_shape=jax.ShapeDtypeStruct((N, D), jnp.float32))
_dense_lsm = pl.pallas_call(
    _dense_lsm_body, out_shape=jax.ShapeDtypeStruct((N, D), jnp.float32))


def kernel(x, edge_index, W_neigh0, W_root0, b0, gamma0, beta0,
           W_neigh1, W_root1, b1, gamma1, beta1, W_neigh2, W_root2, b2):
    src = edge_index[0].astype(jnp.int32)
    dst = edge_index[1].astype(jnp.int32)
    src_slab = jnp.pad(src, (0, EPAD - E)).reshape(NW, CHUNKS, CB)
    dst_slab = jnp.pad(dst, (0, EPAD - E), constant_values=N).reshape(NW, CHUNKS, CB)

    b0r, b1r, b2r = (b.reshape(1, D) for b in (b0, b1, b2))
    g0r, g1r = gamma0.reshape(1, D), gamma1.reshape(1, D)
    be0r, be1r = beta0.reshape(1, D), beta1.reshape(1, D)

    agg_deg = _make_sc_agg(True)
    agg_only = _make_sc_agg(False)

    p0, hist = agg_deg(x, src_slab, dst_slab)
    h1 = _dense_bn(p0, hist, x, W_neigh0, W_root0, b0r, g0r, be0r)
    p1 = agg_only(h1, src_slab, dst_slab)
    h2 = _dense_bn(p1, hist, h1, W_neigh1, W_root1, b1r, g1r, be1r)
    p2 = agg_only(h2, src_slab, dst_slab)
    return _dense_lsm(p2, hist, h2, W_neigh2, W_root2, b2r)
